# baseline (speedup 1.0000x reference)
import jax
import jax.numpy as jnp
from jax.experimental import pallas as pl
from jax.experimental.pallas import tpu as pltpu

N = 256
C = 768
P = 49
D = C * P
PBLK = 7


def _mask_loss_kernel(xl_ref, xr_ref, m_ref, out_ref, s_acc, rs_acc):
    k = pl.program_id(0)

    @pl.when(k == 0)
    def _init():
        s_acc[:, :] = jnp.zeros_like(s_acc)
        rs_acc[:, :] = jnp.zeros_like(rs_acc)

    s = s_acc[:, :]
    rs = rs_acc[:, :]
    for j in range(PBLK):
        for ref in (xl_ref, xr_ref):
            a = ref[j, :, :]
            s = s + jax.lax.dot_general(
                a, a, (((1,), (1,)), ((), ())), preferred_element_type=jnp.float32
            )
            rs = rs + jnp.sum(a, axis=1, keepdims=True)
    s_acc[:, :] = s
    rs_acc[:, :] = rs

    @pl.when(k == P // PBLK - 1)
    def _finalize():
        total = rs_acc[:, 0:1]
        g = s_acc[:, :] - (total * total.T) * (1.0 / D)
        rows = jax.lax.broadcasted_iota(jnp.int32, (N, N), 0)
        cols = jax.lax.broadcasted_iota(jnp.int32, (N, N), 1)
        diag = jnp.sum(jnp.where(rows == cols, g, 0.0), axis=1, keepdims=True)
        inv = jax.lax.rsqrt(diag)
        corr = g * inv * inv.T
        m = m_ref[:, :]
        act = m > 0.0
        masked = jnp.where(act & act.T, corr, 0.0) * (m * m.T)
        out_ref[:, :] = jnp.sqrt(jnp.sum(masked * masked, keepdims=True))


def kernel(weights, mask):
    x = weights.transpose(2, 3, 0, 1).reshape(P, N, C)
    m = mask.reshape(N, 1)
    out = pl.pallas_call(
        _mask_loss_kernel,
        grid=(P // PBLK,),
        in_specs=[
            pl.BlockSpec((PBLK, N, C // 2), lambda k: (k, 0, 0)),
            pl.BlockSpec((PBLK, N, C // 2), lambda k: (k, 0, 1)),
            pl.BlockSpec((N, 1), lambda k: (0, 0)),
        ],
        out_specs=pl.BlockSpec((1, 1), lambda k: (0, 0)),
        out_shape=jax.ShapeDtypeStruct((1, 1), jnp.float32),
        scratch_shapes=[
            pltpu.VMEM((N, N), jnp.float32),
            pltpu.VMEM((N, 1), jnp.float32),
        ],
    )(x, x, m)
    return out[0, 0]


# speedup vs baseline: 1.0217x; 1.0217x over previous
import jax
import jax.numpy as jnp
from jax.experimental import pallas as pl
from jax.experimental.pallas import tpu as pltpu

N = 256
C = 768
P = 49
D = C * P
PBLK = 7


def _mask_loss_kernel(x_ref, m_ref, out_ref, s_acc, rs_acc):
    k = pl.program_id(0)

    @pl.when(k == 0)
    def _init():
        s_acc[:, :] = jnp.zeros_like(s_acc)
        rs_acc[:, :] = jnp.zeros_like(rs_acc)

    s = s_acc[:, :]
    rs = rs_acc[:, :]
    for j in range(PBLK):
        a = x_ref[j, :, :]
        s = s + jax.lax.dot_general(
            a, a, (((1,), (1,)), ((), ())), preferred_element_type=jnp.float32
        )
        rs = rs + jnp.sum(a, axis=1, keepdims=True)
    s_acc[:, :] = s
    rs_acc[:, :] = rs

    @pl.when(k == P // PBLK - 1)
    def _finalize():
        total = rs_acc[:, 0:1]
        g = s_acc[:, :] - (total * total.T) * (1.0 / D)
        rows = jax.lax.broadcasted_iota(jnp.int32, (N, N), 0)
        cols = jax.lax.broadcasted_iota(jnp.int32, (N, N), 1)
        diag = jnp.sum(jnp.where(rows == cols, g, 0.0), axis=1, keepdims=True)
        inv = jax.lax.rsqrt(diag)
        corr = g * inv * inv.T
        m = m_ref[:, :]
        act = m > 0.0
        masked = jnp.where(act & act.T, corr, 0.0) * (m * m.T)
        out_ref[:, :] = jnp.sqrt(jnp.sum(masked * masked, keepdims=True))


def kernel(weights, mask):
    x = weights.transpose(2, 3, 0, 1).reshape(P, N, C)
    m = mask.reshape(N, 1)
    out = pl.pallas_call(
        _mask_loss_kernel,
        grid=(P // PBLK,),
        in_specs=[
            pl.BlockSpec((PBLK, N, C), lambda k: (k, 0, 0)),
            pl.BlockSpec((N, 1), lambda k: (0, 0)),
        ],
        out_specs=pl.BlockSpec((1, 1), lambda k: (0, 0)),
        out_shape=jax.ShapeDtypeStruct((1, 1), jnp.float32),
        scratch_shapes=[
            pltpu.VMEM((N, N), jnp.float32),
            pltpu.VMEM((N, 1), jnp.float32),
        ],
    )(x, m)
    return out[0, 0]


# speedup vs baseline: 1.1242x; 1.1003x over previous
import jax
import jax.numpy as jnp
from jax.experimental import pallas as pl
from jax.experimental.pallas import tpu as pltpu

N = 256
C = 768
P = 49
D = C * P
PBLK = 7


def _mask_loss_kernel(x_ref, m_ref, out_ref, s_acc, rs_acc):
    k = pl.program_id(0)

    @pl.when(k == 0)
    def _init():
        s_acc[:, :] = jnp.zeros_like(s_acc)
        rs_acc[:, :] = jnp.zeros_like(rs_acc)

    s = s_acc[:, :]
    rs = rs_acc[:, :]
    for j in range(PBLK):
        a = x_ref[j, :, :]
        s = s + jax.lax.dot_general(
            a, a, (((1,), (1,)), ((), ())), preferred_element_type=jnp.float32
        )
        rs = rs + jnp.sum(a, axis=1, keepdims=True)
    s_acc[:, :] = s
    rs_acc[:, :] = rs

    @pl.when(k == P // PBLK - 1)
    def _finalize():
        total = rs_acc[:, 0:1]
        g = s_acc[:, :] - (total * total.T) * (1.0 / D)
        rows = jax.lax.broadcasted_iota(jnp.int32, (N, N), 0)
        cols = jax.lax.broadcasted_iota(jnp.int32, (N, N), 1)
        diag = jnp.sum(jnp.where(rows == cols, g, 0.0), axis=1, keepdims=True)
        inv = jax.lax.rsqrt(diag)
        corr = g * inv * inv.T
        mr = m_ref[:, :]
        mc = mr.T
        act = mr > 0.0
        masked = jnp.where(act.T & act, corr, 0.0) * (mc * mr)
        out_ref[:, :] = jnp.sqrt(jnp.sum(masked * masked, keepdims=True))


def kernel(weights, mask):
    x = weights.transpose(2, 3, 0, 1).reshape(P, N, C)
    m = mask.reshape(1, N)
    out = pl.pallas_call(
        _mask_loss_kernel,
        grid=(P // PBLK,),
        in_specs=[
            pl.BlockSpec((PBLK, N, C), lambda k: (k, 0, 0)),
            pl.BlockSpec((1, N), lambda k: (0, 0)),
        ],
        out_specs=pl.BlockSpec((1, 1), lambda k: (0, 0)),
        out_shape=jax.ShapeDtypeStruct((1, 1), jnp.float32),
        scratch_shapes=[
            pltpu.VMEM((N, N), jnp.float32),
            pltpu.VMEM((N, 1), jnp.float32),
        ],
    )(x, m)
    return out[0, 0]
